# baseline (speedup 1.0000x reference)
import jax
import jax.numpy as jnp
from jax import lax
from jax.experimental import pallas as pl
from jax.experimental.pallas import tpu as pltpu
from jax.experimental.pallas import tpu_sc as plsc

VOCAB = 100000
EMBED = 64
BATCH = 16384

NUM_CORES = 2
NUM_SUBCORES = 16
LANES = 16
B_PER_CORE = BATCH // NUM_CORES
D_PER_SUB = EMBED // NUM_SUBCORES
GROUPS = B_PER_CORE // LANES
ACC_ROWS = B_PER_CORE // 128
STRIPE_ROWS = ACC_ROWS // NUM_SUBCORES


def _body(target_hbm, context_hbm, wt_hbm, out_hbm,
          t_idx, c_idx, rowbuf, acc, idxbuf, shared, sem):
    c = lax.axis_index("c")
    s = lax.axis_index("s")
    base = c * B_PER_CORE

    pltpu.sync_copy(target_hbm.at[pl.ds(base, B_PER_CORE)], t_idx)
    pltpu.sync_copy(context_hbm.at[pl.ds(base, B_PER_CORE)], c_idx)

    for k in range(ACC_ROWS // LANES):
        idxbuf[pl.ds(k * LANES, LANES)] = (
            lax.iota(jnp.int32, LANES) + k * LANES)

    for dd in range(D_PER_SUB):
        d = s * D_PER_SUB + dd
        pltpu.sync_copy(wt_hbm.at[d], rowbuf)

        first = dd == 0

        @plsc.parallel_loop(0, GROUPS // 8, unroll=4)
        def _(k):
            for j in range(8):
                sl = pl.ds((k * 8 + j) * LANES, LANES)
                tv = t_idx[sl]
                cv = c_idx[sl]
                tg = plsc.load_gather(rowbuf, [tv])
                cg = plsc.load_gather(rowbuf, [cv])
                prod = tg * cg
                if first:
                    acc[k, pl.ds(j * LANES, LANES)] = prod
                else:
                    plsc.addupdate(acc.at[k, pl.ds(j * LANES, LANES)], prod)

    @pl.when(s == 0)
    def _():
        pltpu.sync_copy(acc, shared)

    plsc.subcore_barrier()

    @pl.when(s != 0)
    def _():
        cp = pltpu.make_async_copy(acc, shared.at[idxbuf], sem)
        cp.start(add=True)
        cp.wait()

    plsc.subcore_barrier()

    rsl = pl.ds(s * STRIPE_ROWS, STRIPE_ROWS)
    osl = pl.ds(c * ACC_ROWS + s * STRIPE_ROWS, STRIPE_ROWS)
    pltpu.sync_copy(shared.at[rsl], out_hbm.at[osl])


@jax.jit
def kernel(target, context, word_embeddings):
    wt = word_embeddings.T
    mesh = plsc.VectorSubcoreMesh(core_axis_name="c", subcore_axis_name="s")
    run = pl.kernel(
        _body,
        out_type=jax.ShapeDtypeStruct((BATCH // 128, 128), jnp.float32),
        mesh=mesh,
        scratch_types=[
            pltpu.VMEM((B_PER_CORE,), jnp.int32),
            pltpu.VMEM((B_PER_CORE,), jnp.int32),
            pltpu.VMEM((VOCAB,), jnp.float32),
            pltpu.VMEM((ACC_ROWS, 128), jnp.float32),
            pltpu.VMEM((ACC_ROWS,), jnp.int32),
            pltpu.VMEM_SHARED((ACC_ROWS, 128), jnp.float32),
            pltpu.SemaphoreType.DMA,
        ],
        compiler_params=pltpu.CompilerParams(
            needs_layout_passes=False, use_tc_tiling_on_sc=True),
    )
    out = run(target, context, wt)
    return jnp.reshape(out, (BATCH,))


# speedup vs baseline: 1.0461x; 1.0461x over previous
import jax
import jax.numpy as jnp
from jax import lax
from jax.experimental import pallas as pl
from jax.experimental.pallas import tpu as pltpu
from jax.experimental.pallas import tpu_sc as plsc

VOCAB = 100000
EMBED = 64
BATCH = 16384

NUM_CORES = 2
NUM_SUBCORES = 16
LANES = 16
B_PER_CORE = BATCH // NUM_CORES
D_PER_SUB = EMBED // NUM_SUBCORES
GROUPS = B_PER_CORE // LANES
ACC_ROWS = B_PER_CORE // 128
STRIPE_ROWS = ACC_ROWS // NUM_SUBCORES


def _body(target_hbm, context_hbm, wt_hbm, out_hbm,
          t_idx, c_idx, rowbuf, acc, idxbuf, shared, sem):
    c = lax.axis_index("c")
    s = lax.axis_index("s")
    base = c * B_PER_CORE

    cps = [
        pltpu.async_copy(target_hbm.at[pl.ds(base, B_PER_CORE)], t_idx, sem),
        pltpu.async_copy(context_hbm.at[pl.ds(base, B_PER_CORE)], c_idx, sem),
        pltpu.async_copy(wt_hbm.at[s * D_PER_SUB], rowbuf, sem),
    ]

    for k in range(ACC_ROWS // LANES):
        idxbuf[pl.ds(k * LANES, LANES)] = (
            lax.iota(jnp.int32, LANES) + k * LANES)

    for cp in cps:
        cp.wait()

    for dd in range(D_PER_SUB):
        d = s * D_PER_SUB + dd
        if dd > 0:
            pltpu.sync_copy(wt_hbm.at[d], rowbuf)

        first = dd == 0

        @plsc.parallel_loop(0, GROUPS // 8, unroll=2)
        def _(k):
            for j in range(8):
                sl = pl.ds((k * 8 + j) * LANES, LANES)
                tv = t_idx[sl]
                cv = c_idx[sl]
                tg = plsc.load_gather(rowbuf, [tv])
                cg = plsc.load_gather(rowbuf, [cv])
                prod = tg * cg
                if first:
                    acc[k, pl.ds(j * LANES, LANES)] = prod
                else:
                    plsc.addupdate(acc.at[k, pl.ds(j * LANES, LANES)], prod)

    @pl.when(s == 0)
    def _():
        pltpu.sync_copy(acc, shared)

    plsc.subcore_barrier()

    @pl.when(s != 0)
    def _():
        cp = pltpu.make_async_copy(acc, shared.at[idxbuf], sem)
        cp.start(add=True)
        cp.wait()

    plsc.subcore_barrier()

    rsl = pl.ds(s * STRIPE_ROWS, STRIPE_ROWS)
    osl = pl.ds(c * ACC_ROWS + s * STRIPE_ROWS, STRIPE_ROWS)
    pltpu.sync_copy(shared.at[rsl], out_hbm.at[osl])


@jax.jit
def kernel(target, context, word_embeddings):
    wt = word_embeddings.T
    mesh = plsc.VectorSubcoreMesh(core_axis_name="c", subcore_axis_name="s")
    run = pl.kernel(
        _body,
        out_type=jax.ShapeDtypeStruct((BATCH // 128, 128), jnp.float32),
        mesh=mesh,
        scratch_types=[
            pltpu.VMEM((B_PER_CORE,), jnp.int32),
            pltpu.VMEM((B_PER_CORE,), jnp.int32),
            pltpu.VMEM((VOCAB,), jnp.float32),
            pltpu.VMEM((ACC_ROWS, 128), jnp.float32),
            pltpu.VMEM((ACC_ROWS,), jnp.int32),
            pltpu.VMEM_SHARED((ACC_ROWS, 128), jnp.float32),
            pltpu.SemaphoreType.DMA,
        ],
        compiler_params=pltpu.CompilerParams(
            needs_layout_passes=False, use_tc_tiling_on_sc=True),
    )
    out = run(target, context, wt)
    return jnp.reshape(out, (BATCH,))
